# baseline (speedup 1.0000x reference)
import functools

import jax
import jax.numpy as jnp
from jax import lax
from jax.experimental import pallas as pl
from jax.experimental.pallas import tpu as pltpu
from jax.experimental.pallas import tpu_sc as plsc

B_TOK = 16384
S_TOK = 50
D = 64
NUM_EMB = 1000000
B = B_TOK * S_TOK

_info = plsc.get_sparse_core_info()
NC = _info.num_cores
NS = _info.num_subcores
NW = NC * NS
LB = 128
GROUP = 512
BLOCKS_PER_GROUP = GROUP // LB
N_BLOCKS = B // LB
N_GROUPS = B // GROUP
G_PER_W = N_GROUPS // NW


def _k0_flatten(ids_t_hbm, out_hbm, vbuf):
    wid = lax.axis_index("s") * NC + lax.axis_index("c")
    for rep in range(2):
        s = wid + NW * rep

        @pl.when(s < S_TOK)
        def _():
            pltpu.sync_copy(ids_t_hbm.at[s], vbuf)
            pltpu.sync_copy(vbuf, out_hbm.at[pl.ds(s * B_TOK, B_TOK)])


T_BLK = 128
N_FULL = NUM_EMB // T_BLK
TAIL0 = N_FULL * T_BLK
N_TAIL = NUM_EMB - TAIL0
K1_SLOTS = 246


def _k1_repack(wt_hbm, out_hbm, tin, tpad, tout, isem, osem):
    wid = lax.axis_index("s") * NC + lax.axis_index("c")
    iota16 = jnp.arange(16, dtype=jnp.int32)
    fidx = [iota16 + 16 * c for c in range(4)]

    def blk_id(k):
        return jnp.minimum(wid + NW * k, N_FULL - 1)

    def start_in(k, buf):
        b = blk_id(k)
        for fg in range(8):
            pltpu.async_copy(
                wt_hbm.at[pl.ds(8 * fg, 8), pl.ds(T_BLK * b, T_BLK)],
                tin.at[buf, fg], isem.at[buf])

    def wait_in(buf):
        for fg in range(8):
            pltpu.make_async_copy(
                wt_hbm.at[pl.ds(0, 8), pl.ds(0, T_BLK)],
                tin.at[buf, fg], isem.at[buf]).wait()

    def out_descr(buf):
        return pltpu.make_async_copy(tout.at[buf],
                                     out_hbm.at[pl.ds(0, T_BLK)],
                                     osem.at[buf])

    start_in(0, 0)
    start_in(1, 1)

    def slot(k, buf):
        wait_in(buf)

        @plsc.parallel_loop(0, D, unroll=4)
        def _(d):
            fg = d // 8
            sub = d % 8
            for c in range(8):
                tpad[d, pl.ds(16 * c, 16)] = tin[buf, fg, sub,
                                                 pl.ds(16 * c, 16)]

        @pl.when(k + 2 < K1_SLOTS)
        def _():
            start_in(k + 2, buf)

        @pl.when(k >= 2)
        def _():
            out_descr(buf).wait()

        @plsc.parallel_loop(0, T_BLK, unroll=4)
        def _(l):
            lsplat = jnp.full((16,), 0, dtype=jnp.int32) + l
            for c in range(4):
                v = plsc.load_gather(tpad, [fidx[c], lsplat])
                tout[buf, l, pl.ds(16 * c, 16)] = v

        pltpu.async_copy(tout.at[buf],
                         out_hbm.at[pl.ds(T_BLK * blk_id(k), T_BLK)],
                         osem.at[buf])

    def outer(h, carry):
        for sub in range(2):
            slot(2 * h + sub, sub)
        return carry

    lax.fori_loop(0, K1_SLOTS // 2, outer, 0)

    for buf in range(2):
        out_descr(buf).wait()

    @pl.when(wid == 0)
    def _():
        for d in range(D):
            pltpu.async_copy(wt_hbm.at[d, pl.ds(TAIL0, N_TAIL)],
                             tin.at[0, d // 8, d % 8, pl.ds(0, N_TAIL)],
                             isem.at[0])
        for d in range(D):
            pltpu.make_async_copy(wt_hbm.at[0, pl.ds(0, N_TAIL)],
                                  tin.at[0, 0, 0, pl.ds(0, N_TAIL)],
                                  isem.at[0]).wait()

        @plsc.parallel_loop(0, D, unroll=4)
        def _(d):
            fg = d // 8
            sub = d % 8
            for c in range(4):
                tpad[d, pl.ds(16 * c, 16)] = tin[0, fg, sub,
                                                 pl.ds(16 * c, 16)]

        @plsc.parallel_loop(0, N_TAIL, unroll=4)
        def _(l):
            lsplat = jnp.full((16,), 0, dtype=jnp.int32) + l
            for c in range(4):
                v = plsc.load_gather(tpad, [fidx[c], lsplat])
                tout[0, l, pl.ds(16 * c, 16)] = v

        pltpu.sync_copy(tout.at[0, pl.ds(0, N_TAIL)],
                        out_hbm.at[pl.ds(TAIL0, N_TAIL)])


def _k2_gather(table_hbm, ids2_hbm, out_hbm, idx_all, rows_v, rows_pad,
               rowsT, gsem, ssem):
    wid = lax.axis_index("s") * NC + lax.axis_index("c")
    g0 = G_PER_W * wid

    iota16 = jnp.arange(16, dtype=jnp.int32)
    lidx = [iota16 + 16 * c for c in range(8)]

    pltpu.sync_copy(ids2_hbm.at[pl.ds(g0, G_PER_W)], idx_all)

    def start_gather(g, b):
        pltpu.async_copy(table_hbm.at[idx_all.at[g]], rows_v.at[b],
                         gsem.at[b])

    def gather_descr(b):
        return pltpu.make_async_copy(table_hbm.at[idx_all.at[0]],
                                     rows_v.at[b], gsem.at[b])

    def store_descr(tb):
        return pltpu.make_async_copy(
            rowsT.at[tb],
            out_hbm.at[pl.ds(0, 1), slice(None), pl.ds(0, 1)],
            ssem.at[tb])

    def transpose_block(b, blk, tb):
        @plsc.parallel_loop(0, LB, unroll=4)
        def _(l):
            for c in range(4):
                rows_pad[l, pl.ds(16 * c, 16)] = (
                    rows_v[b, LB * blk + l, pl.ds(16 * c, 16)])

        @plsc.parallel_loop(0, D, unroll=4)
        def _(d):
            dsplat = jnp.full((16,), 0, dtype=jnp.int32) + d
            fg = d // 8
            off = (d % 8) * 128
            for c in range(8):
                v = plsc.load_gather(rows_pad, [lidx[c], dsplat])
                rowsT[tb, 0, fg, 0, pl.ds(off + 16 * c, 16)] = v

    def store_block(g, blk, tb):
        gid = BLOCKS_PER_GROUP * (g0 + g) + blk
        s = gid // 128
        bt = gid % 128
        pltpu.async_copy(
            rowsT.at[tb],
            out_hbm.at[pl.ds(s, 1), slice(None), pl.ds(bt, 1)],
            ssem.at[tb])

    start_gather(0, 0)

    def outer(h, carry):
        for sub in range(2):
            g = 2 * h + sub
            b = sub
            gather_descr(b).wait()

            @pl.when(g + 1 < G_PER_W)
            def _():
                start_gather(g + 1, 1 - b)

            for blk in range(BLOCKS_PER_GROUP):
                tb = blk % 2
                if blk >= 2:
                    store_descr(tb).wait()
                else:
                    @pl.when(g > 0)
                    def _():
                        store_descr(tb).wait()
                transpose_block(b, blk, tb)
                store_block(g, blk, tb)
        return carry

    lax.fori_loop(0, G_PER_W // 2, outer, 0)

    for tb in range(2):
        store_descr(tb).wait()


def kernel(token_ids, weight):
    mesh = plsc.VectorSubcoreMesh(core_axis_name="c", subcore_axis_name="s")

    ids_t = token_ids.T

    k0 = functools.partial(
        pl.kernel,
        mesh=mesh,
        out_type=jax.ShapeDtypeStruct((B,), jnp.int32),
        scratch_types=[pltpu.VMEM((B_TOK,), jnp.int32)],
    )(_k0_flatten)
    ids_flat = k0(ids_t.astype(jnp.int32))
    ids2 = ids_flat.reshape(N_GROUPS, GROUP)

    k1 = functools.partial(
        pl.kernel,
        mesh=mesh,
        out_type=jax.ShapeDtypeStruct((NUM_EMB, D), jnp.float32),
        scratch_types=[
            pltpu.VMEM((2, 8, 8, T_BLK), jnp.float32),
            pltpu.VMEM((D, 129), jnp.float32),
            pltpu.VMEM((2, T_BLK, D), jnp.float32),
            pltpu.SemaphoreType.DMA((2,)),
            pltpu.SemaphoreType.DMA((2,)),
        ],
        compiler_params=pltpu.CompilerParams(needs_layout_passes=False),
    )(_k1_repack)
    w_lin = k1(weight.T)

    k2 = functools.partial(
        pl.kernel,
        mesh=mesh,
        out_type=jax.ShapeDtypeStruct((S_TOK, 8, 128, 1024), jnp.float32),
        scratch_types=[
            pltpu.VMEM((G_PER_W, GROUP), jnp.int32),
            pltpu.VMEM((2, GROUP, D), jnp.float32),
            pltpu.VMEM((LB, 65), jnp.float32),
            pltpu.VMEM((2, 1, 8, 1, 1024), jnp.float32),
            pltpu.SemaphoreType.DMA((2,)),
            pltpu.SemaphoreType.DMA((2,)),
        ],
        compiler_params=pltpu.CompilerParams(use_tc_tiling_on_sc=False,
                                             needs_layout_passes=False),
    )(_k2_gather)
    out4 = k2(w_lin, ids2)

    out5 = out4.reshape(S_TOK, 8, 128, 8, 128)
    return jnp.transpose(out5, (2, 4, 0, 1, 3)).reshape(B_TOK, S_TOK, D)


# speedup vs baseline: 1.3562x; 1.3562x over previous
import functools

import jax
import jax.numpy as jnp
from jax import lax
from jax.experimental import pallas as pl
from jax.experimental.pallas import tpu as pltpu
from jax.experimental.pallas import tpu_sc as plsc

B_TOK = 16384
S_TOK = 50
D = 64
NUM_EMB = 1000000
B = B_TOK * S_TOK

_info = plsc.get_sparse_core_info()
NC = _info.num_cores
NS = _info.num_subcores
NW = NC * NS
LB = 128
GROUP = 512
BLOCKS_PER_GROUP = GROUP // LB
N_BLOCKS = B // LB
N_GROUPS = B // GROUP
G_PER_W = N_GROUPS // NW


def _k0_flatten(ids_t_hbm, out_hbm, vbuf):
    wid = lax.axis_index("s") * NC + lax.axis_index("c")
    for rep in range(2):
        s = wid + NW * rep

        @pl.when(s < S_TOK)
        def _():
            pltpu.sync_copy(ids_t_hbm.at[s], vbuf)
            pltpu.sync_copy(vbuf, out_hbm.at[pl.ds(s * B_TOK, B_TOK)])


T_BLK = 128
N_FULL = NUM_EMB // T_BLK
TAIL0 = N_FULL * T_BLK
N_TAIL = NUM_EMB - TAIL0
K1_SLOTS = 246


def _k1_repack(wt_hbm, out_hbm, tpad, tout, isem, osem):
    wid = lax.axis_index("s") * NC + lax.axis_index("c")
    iota16 = jnp.arange(16, dtype=jnp.int32)
    fidx = [iota16 + 16 * c for c in range(4)]

    def blk_id(k):
        return jnp.minimum(wid + NW * k, N_FULL - 1)

    def start_in(k, buf):
        b = blk_id(k)
        for fg in range(8):
            pltpu.async_copy(
                wt_hbm.at[pl.ds(8 * fg, 8), pl.ds(T_BLK * b, T_BLK)],
                tpad.at[buf, pl.ds(8 * fg, 8), pl.ds(0, T_BLK)],
                isem.at[buf])

    def wait_in(buf):
        for fg in range(8):
            pltpu.make_async_copy(
                wt_hbm.at[pl.ds(0, 8), pl.ds(0, T_BLK)],
                tpad.at[buf, pl.ds(0, 8), pl.ds(0, T_BLK)],
                isem.at[buf]).wait()

    def out_descr(buf):
        return pltpu.make_async_copy(tout.at[buf],
                                     out_hbm.at[pl.ds(0, T_BLK * D)],
                                     osem.at[buf])

    start_in(0, 0)
    start_in(1, 1)

    def slot(k, buf):
        wait_in(buf)

        @pl.when(k >= 2)
        def _():
            out_descr(buf).wait()

        @plsc.parallel_loop(0, T_BLK, unroll=4)
        def _(l):
            lsplat = jnp.full((16,), 0, dtype=jnp.int32) + l
            for c in range(4):
                v = plsc.load_gather(tpad.at[buf], [fidx[c], lsplat])
                tout[buf, pl.ds(D * l + 16 * c, 16)] = v

        @pl.when(k + 2 < K1_SLOTS)
        def _():
            start_in(k + 2, buf)

        pltpu.async_copy(tout.at[buf],
                         out_hbm.at[pl.ds(T_BLK * D * blk_id(k), T_BLK * D)],
                         osem.at[buf])

    def outer(h, carry):
        for sub in range(2):
            slot(2 * h + sub, sub)
        return carry

    lax.fori_loop(0, K1_SLOTS // 2, outer, 0)

    for buf in range(2):
        out_descr(buf).wait()

    @pl.when(wid == 0)
    def _():
        for d in range(D):
            pltpu.async_copy(wt_hbm.at[d, pl.ds(TAIL0, N_TAIL)],
                             tpad.at[0, d, pl.ds(0, N_TAIL)],
                             isem.at[0])
        for d in range(D):
            pltpu.make_async_copy(wt_hbm.at[0, pl.ds(0, N_TAIL)],
                                  tpad.at[0, 0, pl.ds(0, N_TAIL)],
                                  isem.at[0]).wait()

        @plsc.parallel_loop(0, N_TAIL, unroll=4)
        def _(l):
            lsplat = jnp.full((16,), 0, dtype=jnp.int32) + l
            for c in range(4):
                v = plsc.load_gather(tpad.at[0], [fidx[c], lsplat])
                tout[0, pl.ds(D * l + 16 * c, 16)] = v

        pltpu.sync_copy(tout.at[0, pl.ds(0, N_TAIL * D)],
                        out_hbm.at[pl.ds(TAIL0 * D, N_TAIL * D)])


def _k2_gather(table_hbm, ids2_hbm, out_hbm, idx_all, rows_v, rows_pad,
               rowsT, gsem, ssem):
    wid = lax.axis_index("s") * NC + lax.axis_index("c")
    g0 = G_PER_W * wid

    iota16 = jnp.arange(16, dtype=jnp.int32)
    lidx = [iota16 + 16 * c for c in range(8)]

    pltpu.sync_copy(ids2_hbm.at[pl.ds(g0, G_PER_W)], idx_all)

    def start_gather(g, b):
        pltpu.async_copy(table_hbm.at[idx_all.at[g]], rows_v.at[b],
                         gsem.at[b])

    def gather_descr(b):
        return pltpu.make_async_copy(table_hbm.at[idx_all.at[0]],
                                     rows_v.at[b], gsem.at[b])

    def store_descr(tb):
        return pltpu.make_async_copy(
            rowsT.at[tb],
            out_hbm.at[pl.ds(0, 1), slice(None), pl.ds(0, 1)],
            ssem.at[tb])

    def transpose_block(b, blk, tb):
        @plsc.parallel_loop(0, LB, unroll=4)
        def _(l):
            for c in range(4):
                rows_pad[l, pl.ds(16 * c, 16)] = (
                    rows_v[b, LB * blk + l, pl.ds(16 * c, 16)])

        @plsc.parallel_loop(0, D, unroll=4)
        def _(d):
            dsplat = jnp.full((16,), 0, dtype=jnp.int32) + d
            fg = d // 8
            off = (d % 8) * 128
            for c in range(8):
                v = plsc.load_gather(rows_pad, [lidx[c], dsplat])
                rowsT[tb, 0, fg, 0, pl.ds(off + 16 * c, 16)] = v

    def store_block(g, blk, tb):
        gid = BLOCKS_PER_GROUP * (g0 + g) + blk
        s = gid // 128
        bt = gid % 128
        pltpu.async_copy(
            rowsT.at[tb],
            out_hbm.at[pl.ds(s, 1), slice(None), pl.ds(bt, 1)],
            ssem.at[tb])

    start_gather(0, 0)

    def outer(h, carry):
        for sub in range(2):
            g = 2 * h + sub
            b = sub
            gather_descr(b).wait()

            @pl.when(g + 1 < G_PER_W)
            def _():
                start_gather(g + 1, 1 - b)

            for blk in range(BLOCKS_PER_GROUP):
                tb = blk % 2
                if blk >= 2:
                    store_descr(tb).wait()
                else:
                    @pl.when(g > 0)
                    def _():
                        store_descr(tb).wait()
                transpose_block(b, blk, tb)
                store_block(g, blk, tb)
        return carry

    lax.fori_loop(0, G_PER_W // 2, outer, 0)

    for tb in range(2):
        store_descr(tb).wait()


def kernel(token_ids, weight):
    mesh = plsc.VectorSubcoreMesh(core_axis_name="c", subcore_axis_name="s")

    ids_t = token_ids.T

    k0 = functools.partial(
        pl.kernel,
        mesh=mesh,
        out_type=jax.ShapeDtypeStruct((B,), jnp.int32),
        scratch_types=[pltpu.VMEM((B_TOK,), jnp.int32)],
    )(_k0_flatten)
    ids_flat = k0(ids_t.astype(jnp.int32))
    ids2 = ids_flat.reshape(N_GROUPS, GROUP)

    k1 = functools.partial(
        pl.kernel,
        mesh=mesh,
        out_type=jax.ShapeDtypeStruct((NUM_EMB * D,), jnp.float32),
        scratch_types=[
            pltpu.VMEM((2, D, 129), jnp.float32),
            pltpu.VMEM((2, T_BLK * D), jnp.float32),
            pltpu.SemaphoreType.DMA((2,)),
            pltpu.SemaphoreType.DMA((2,)),
        ],
        compiler_params=pltpu.CompilerParams(needs_layout_passes=False),
    )(_k1_repack)
    w_lin = k1(weight.T).reshape(NUM_EMB, D)

    k2 = functools.partial(
        pl.kernel,
        mesh=mesh,
        out_type=jax.ShapeDtypeStruct((S_TOK, 8, 128, 1024), jnp.float32),
        scratch_types=[
            pltpu.VMEM((G_PER_W, GROUP), jnp.int32),
            pltpu.VMEM((2, GROUP, D), jnp.float32),
            pltpu.VMEM((LB, 65), jnp.float32),
            pltpu.VMEM((2, 1, 8, 1, 1024), jnp.float32),
            pltpu.SemaphoreType.DMA((2,)),
            pltpu.SemaphoreType.DMA((2,)),
        ],
        compiler_params=pltpu.CompilerParams(use_tc_tiling_on_sc=False,
                                             needs_layout_passes=False),
    )(_k2_gather)
    out4 = k2(w_lin, ids2)

    out5 = out4.reshape(S_TOK, 8, 128, 8, 128)
    return jnp.transpose(out5, (2, 4, 0, 1, 3)).reshape(B_TOK, S_TOK, D)


# speedup vs baseline: 1.8755x; 1.3829x over previous
import functools

import jax
import jax.numpy as jnp
from jax import lax
from jax.experimental import pallas as pl
from jax.experimental.pallas import tpu as pltpu
from jax.experimental.pallas import tpu_sc as plsc

B_TOK = 16384
S_TOK = 50
D = 64
NUM_EMB = 1000000
B = B_TOK * S_TOK

_info = plsc.get_sparse_core_info()
NC = _info.num_cores
NS = _info.num_subcores
NW = NC * NS
LB = 128
GROUP = 512
BLOCKS_PER_GROUP = GROUP // LB
N_BLOCKS = B // LB
N_GROUPS = B // GROUP
G_PER_W = N_GROUPS // NW


def _k0_flatten(ids_t_hbm, out_hbm, vbuf):
    wid = lax.axis_index("s") * NC + lax.axis_index("c")
    for rep in range(2):
        s = wid + NW * rep

        @pl.when(s < S_TOK)
        def _():
            pltpu.sync_copy(ids_t_hbm.at[s], vbuf)
            pltpu.sync_copy(vbuf, out_hbm.at[pl.ds(s * B_TOK, B_TOK)])


def _k2_gather(table_hbm, ids2_hbm, out_hbm, idx_all, rows_v, rows_pad,
               rowsT, gsem, ssem):
    wid = lax.axis_index("s") * NC + lax.axis_index("c")
    g0 = G_PER_W * wid

    iota16 = jnp.arange(16, dtype=jnp.int32)
    lidx = [iota16 + 16 * c for c in range(8)]

    pltpu.sync_copy(ids2_hbm.at[pl.ds(g0, G_PER_W)], idx_all)

    def start_gather(g, b):
        pltpu.async_copy(table_hbm.at[idx_all.at[g]], rows_v.at[b],
                         gsem.at[b])

    def gather_descr(b):
        return pltpu.make_async_copy(table_hbm.at[idx_all.at[0]],
                                     rows_v.at[b], gsem.at[b])

    def store_descr(tb):
        return pltpu.make_async_copy(
            rowsT.at[tb],
            out_hbm.at[pl.ds(0, 1), slice(None), pl.ds(0, 1)],
            ssem.at[tb])

    def transpose_block(b, blk, tb):
        @plsc.parallel_loop(0, LB, unroll=4)
        def _(l):
            for c in range(4):
                rows_pad[l, pl.ds(16 * c, 16)] = (
                    rows_v[b, LB * blk + l, pl.ds(16 * c, 16)])

        @plsc.parallel_loop(0, D, unroll=4)
        def _(d):
            dsplat = jnp.full((16,), 0, dtype=jnp.int32) + d
            fg = d // 8
            off = (d % 8) * 128
            for c in range(8):
                v = plsc.load_gather(rows_pad, [lidx[c], dsplat])
                rowsT[tb, 0, fg, 0, pl.ds(off + 16 * c, 16)] = v

    def store_block(g, blk, tb):
        gid = BLOCKS_PER_GROUP * (g0 + g) + blk
        s = gid // 128
        bt = gid % 128
        pltpu.async_copy(
            rowsT.at[tb],
            out_hbm.at[pl.ds(s, 1), slice(None), pl.ds(bt, 1)],
            ssem.at[tb])

    start_gather(0, 0)

    def outer(h, carry):
        for sub in range(2):
            g = 2 * h + sub
            b = sub
            gather_descr(b).wait()

            @pl.when(g + 1 < G_PER_W)
            def _():
                start_gather(g + 1, 1 - b)

            for blk in range(BLOCKS_PER_GROUP):
                tb = blk % 2
                if blk >= 2:
                    store_descr(tb).wait()
                else:
                    @pl.when(g > 0)
                    def _():
                        store_descr(tb).wait()
                transpose_block(b, blk, tb)
                store_block(g, blk, tb)
        return carry

    lax.fori_loop(0, G_PER_W // 2, outer, 0)

    for tb in range(2):
        store_descr(tb).wait()


def kernel(token_ids, weight):
    mesh = plsc.VectorSubcoreMesh(core_axis_name="c", subcore_axis_name="s")

    ids_t = token_ids.T

    k0 = functools.partial(
        pl.kernel,
        mesh=mesh,
        out_type=jax.ShapeDtypeStruct((B,), jnp.int32),
        scratch_types=[pltpu.VMEM((B_TOK,), jnp.int32)],
    )(_k0_flatten)
    ids_flat = k0(ids_t.astype(jnp.int32))
    ids2 = ids_flat.reshape(N_GROUPS, GROUP)


    k2 = functools.partial(
        pl.kernel,
        mesh=mesh,
        out_type=jax.ShapeDtypeStruct((S_TOK, 8, 128, 1024), jnp.float32),
        scratch_types=[
            pltpu.VMEM((G_PER_W, GROUP), jnp.int32),
            pltpu.VMEM((2, GROUP, D), jnp.float32),
            pltpu.VMEM((LB, 65), jnp.float32),
            pltpu.VMEM((2, 1, 8, 1, 1024), jnp.float32),
            pltpu.SemaphoreType.DMA((2,)),
            pltpu.SemaphoreType.DMA((2,)),
        ],
        compiler_params=pltpu.CompilerParams(use_tc_tiling_on_sc=False,
                                             needs_layout_passes=False),
    )(_k2_gather)
    out4 = k2(weight, ids2)

    out5 = out4.reshape(S_TOK, 8, 128, 8, 128)
    return jnp.transpose(out5, (2, 4, 0, 1, 3)).reshape(B_TOK, S_TOK, D)
